# baseline (speedup 1.0000x reference)
import functools

import jax
import jax.numpy as jnp
from jax import lax
from jax.experimental import pallas as pl
from jax.experimental.pallas import tpu as pltpu
from jax.experimental.pallas import tpu_sc as plsc

L = 16
NW = 32
D = 256
DV = D // L
C = 64
PB = 32


def _decoder_body(E, z_hbm, src_hbm, dst_hbm, out_hbm,
                  sidx_v, didx_v, s0_v, d0_v, s1_v, d1_v, pbuf_v, outv_v,
                  ss0, sd0, ss1, sd1):
    nchunk = E // C
    bnk = nchunk // NW
    rem = nchunk - bnk * NW
    maxnk = bnk + (1 if rem else 0)
    wid = lax.axis_index("c") * 16 + lax.axis_index("s")
    nk = jnp.where(wid < rem, bnk + 1, bnk)
    start_chunk = wid * bnk + jnp.minimum(wid, rem)
    ebase = start_chunk * C

    pltpu.sync_copy(src_hbm.at[pl.ds(ebase, maxnk * C)], sidx_v)
    pltpu.sync_copy(dst_hbm.at[pl.ds(ebase, maxnk * C)], didx_v)

    def start(c, sbuf, dbuf, ssem, dsem):
        pltpu.async_copy(z_hbm.at[sidx_v.at[pl.ds(c * C, C)]], sbuf, ssem)
        pltpu.async_copy(z_hbm.at[didx_v.at[pl.ds(c * C, C)]], dbuf, dsem)

    def wait(c, sbuf, dbuf, ssem, dsem):
        pltpu.make_async_copy(
            z_hbm.at[sidx_v.at[pl.ds(c * C, C)]], sbuf, ssem).wait()
        pltpu.make_async_copy(
            z_hbm.at[didx_v.at[pl.ds(c * C, C)]], dbuf, dsem).wait()

    lanes = lax.broadcasted_iota(jnp.int32, (L,), 0)
    maskf = [jnp.where(lanes == e, 1.0, 0.0).astype(jnp.float32)
             for e in range(L)]

    def compute(c, sbuf, dbuf):
        for g in range(C // L):
            for e in range(L):
                row = g * L + e
                acc = (sbuf[row, pl.ds(0, L)] * dbuf[row, pl.ds(0, L)])
                for i in range(1, DV):
                    acc = acc + (sbuf[row, pl.ds(i * L, L)]
                                 * dbuf[row, pl.ds(i * L, L)])
                pbuf_v[pl.ds(e * PB, L)] = acc
                for h in (8, 4, 2):
                    acc = acc + pbuf_v[pl.ds(e * PB + h, L)]
                    pbuf_v[pl.ds(e * PB, L)] = acc
            t = [(pbuf_v[pl.ds((PB - 1) * e, L)]
                  + pbuf_v[pl.ds((PB - 1) * e + 1, L)]) * maskf[e]
                 for e in range(L)]
            while len(t) > 1:
                t = [t[i] + t[i + 1] for i in range(0, len(t), 2)]
            outv_v[pl.ds(g * L, L)] = 1.0 / (1.0 + jnp.exp(-t[0]))
        pltpu.sync_copy(outv_v, out_hbm.at[pl.ds(ebase + c * C, C)])

    start(0, s0_v, d0_v, ss0, sd0)

    def pipe_body(kk, _):
        c0 = 2 * kk
        c1 = c0 + 1
        c2 = c0 + 2

        @pl.when(c1 < nk)
        def _():
            start(c1, s1_v, d1_v, ss1, sd1)

        @pl.when(c0 < nk)
        def _():
            wait(c0, s0_v, d0_v, ss0, sd0)
            compute(c0, s0_v, d0_v)

        @pl.when(c2 < nk)
        def _():
            start(c2, s0_v, d0_v, ss0, sd0)

        @pl.when(c1 < nk)
        def _():
            wait(c1, s1_v, d1_v, ss1, sd1)
            compute(c1, s1_v, d1_v)

        return 0

    lax.fori_loop(0, (maxnk + 1) // 2, pipe_body, 0)


def kernel(z, edge_index):
    E = edge_index.shape[1]
    nchunk = E // C
    bnk = nchunk // NW
    maxnk = bnk + (1 if nchunk % NW else 0)
    pad = maxnk * C * NW - E + C
    src = jnp.pad(edge_index[0], (0, pad))
    dst = jnp.pad(edge_index[1], (0, pad))

    mesh = plsc.VectorSubcoreMesh(core_axis_name="c", subcore_axis_name="s")
    body = functools.partial(_decoder_body, E)
    f = pl.kernel(
        body,
        out_type=jax.ShapeDtypeStruct((E,), jnp.float32),
        mesh=mesh,
        scratch_types=[
            pltpu.VMEM((maxnk * C,), jnp.int32),
            pltpu.VMEM((maxnk * C,), jnp.int32),
            pltpu.VMEM((C, D), jnp.float32),
            pltpu.VMEM((C, D), jnp.float32),
            pltpu.VMEM((C, D), jnp.float32),
            pltpu.VMEM((C, D), jnp.float32),
            pltpu.VMEM((L * PB + L,), jnp.float32),
            pltpu.VMEM((C,), jnp.float32),
            pltpu.SemaphoreType.DMA,
            pltpu.SemaphoreType.DMA,
            pltpu.SemaphoreType.DMA,
            pltpu.SemaphoreType.DMA,
        ],
    )
    return f(z, src, dst)


# speedup vs baseline: 2.0713x; 2.0713x over previous
import functools

import jax
import jax.numpy as jnp
from jax import lax
from jax.experimental import pallas as pl
from jax.experimental.pallas import tpu as pltpu
from jax.experimental.pallas import tpu_sc as plsc

L = 16
NW = 32
D = 256
DV = D // L
C = 64
PB = 32


def _decoder_body(E, z_hbm, src_hbm, dst_hbm, out_hbm,
                  sidx_v, didx_v, s0_v, d0_v, pbuf_v, outv_v,
                  ss0, sd0, ss1, sd1):
    nchunk = E // C
    bnk = nchunk // NW
    rem = nchunk - bnk * NW
    maxnk = bnk + (1 if rem else 0)
    wid = lax.axis_index("c") * 16 + lax.axis_index("s")
    nk = jnp.where(wid < rem, bnk + 1, bnk)
    start_chunk = wid * bnk + jnp.minimum(wid, rem)
    ebase = start_chunk * C

    pltpu.sync_copy(src_hbm.at[pl.ds(ebase, maxnk * C)], sidx_v)
    pltpu.sync_copy(dst_hbm.at[pl.ds(ebase, maxnk * C)], didx_v)

    def start(c):
        par = c % 2

        @pl.when(par == 0)
        def _():
            pltpu.async_copy(z_hbm.at[sidx_v.at[pl.ds(c * C, C)]],
                             s0_v.at[pl.ds(0, C)], ss0)
            pltpu.async_copy(z_hbm.at[didx_v.at[pl.ds(c * C, C)]],
                             d0_v.at[pl.ds(0, C)], sd0)

        @pl.when(par == 1)
        def _():
            pltpu.async_copy(z_hbm.at[sidx_v.at[pl.ds(c * C, C)]],
                             s0_v.at[pl.ds(C, C)], ss1)
            pltpu.async_copy(z_hbm.at[didx_v.at[pl.ds(c * C, C)]],
                             d0_v.at[pl.ds(C, C)], sd1)

    def wait(c):
        par = c % 2

        @pl.when(par == 0)
        def _():
            pltpu.make_async_copy(z_hbm.at[sidx_v.at[pl.ds(c * C, C)]],
                                  s0_v.at[pl.ds(0, C)], ss0).wait()
            pltpu.make_async_copy(z_hbm.at[didx_v.at[pl.ds(c * C, C)]],
                                  d0_v.at[pl.ds(0, C)], sd0).wait()

        @pl.when(par == 1)
        def _():
            pltpu.make_async_copy(z_hbm.at[sidx_v.at[pl.ds(c * C, C)]],
                                  s0_v.at[pl.ds(C, C)], ss1).wait()
            pltpu.make_async_copy(z_hbm.at[didx_v.at[pl.ds(c * C, C)]],
                                  d0_v.at[pl.ds(C, C)], sd1).wait()

    lanes = lax.broadcasted_iota(jnp.int32, (L,), 0)
    maskf = [jnp.where(lanes == e, 1.0, 0.0).astype(jnp.float32)
             for e in range(L)]

    def compute(c):
        rowb = (c % 2) * C

        def group_body(g, _):
            for e in range(L):
                row = rowb + g * L + e
                acc = (s0_v[row, pl.ds(0, L)] * d0_v[row, pl.ds(0, L)])
                for i in range(1, DV):
                    acc = acc + (s0_v[row, pl.ds(i * L, L)]
                                 * d0_v[row, pl.ds(i * L, L)])
                pbuf_v[pl.ds(e * PB, L)] = acc
                for h in (8, 4, 2):
                    acc = acc + pbuf_v[pl.ds(e * PB + h, L)]
                    pbuf_v[pl.ds(e * PB, L)] = acc
            t = [(pbuf_v[pl.ds((PB - 1) * e, L)]
                  + pbuf_v[pl.ds((PB - 1) * e + 1, L)]) * maskf[e]
                 for e in range(L)]
            while len(t) > 1:
                t = [t[i] + t[i + 1] for i in range(0, len(t), 2)]
            outv_v[pl.ds(g * L, L)] = 1.0 / (1.0 + jnp.exp(-t[0]))
            return 0

        lax.fori_loop(0, C // L, group_body, 0)
        pltpu.sync_copy(outv_v, out_hbm.at[pl.ds(ebase + c * C, C)])

    start(0)

    def pipe_body(c, _):
        @pl.when(c + 1 < nk)
        def _():
            start(c + 1)

        @pl.when(c < nk)
        def _():
            wait(c)
            compute(c)

        return 0

    lax.fori_loop(0, maxnk, pipe_body, 0)


def kernel(z, edge_index):
    E = edge_index.shape[1]
    nchunk = E // C
    bnk = nchunk // NW
    maxnk = bnk + (1 if nchunk % NW else 0)
    pad = maxnk * C * NW - E + C
    src = jnp.pad(edge_index[0], (0, pad))
    dst = jnp.pad(edge_index[1], (0, pad))

    mesh = plsc.VectorSubcoreMesh(core_axis_name="c", subcore_axis_name="s")
    body = functools.partial(_decoder_body, E)
    f = pl.kernel(
        body,
        out_type=jax.ShapeDtypeStruct((E,), jnp.float32),
        mesh=mesh,
        scratch_types=[
            pltpu.VMEM((maxnk * C,), jnp.int32),
            pltpu.VMEM((maxnk * C,), jnp.int32),
            pltpu.VMEM((2 * C, D), jnp.float32),
            pltpu.VMEM((2 * C, D), jnp.float32),
            pltpu.VMEM((L * PB + L,), jnp.float32),
            pltpu.VMEM((C,), jnp.float32),
            pltpu.SemaphoreType.DMA,
            pltpu.SemaphoreType.DMA,
            pltpu.SemaphoreType.DMA,
            pltpu.SemaphoreType.DMA,
        ],
    )
    return f(z, src, dst)


# speedup vs baseline: 2.0855x; 1.0068x over previous
import functools

import jax
import jax.numpy as jnp
from jax import lax
from jax.experimental import pallas as pl
from jax.experimental.pallas import tpu as pltpu
from jax.experimental.pallas import tpu_sc as plsc

L = 16
NW = 32
D = 256
DV = D // L
C = 64
PB = 32


def _decoder_body(E, z_hbm, src_hbm, dst_hbm, out_hbm,
                  sidx_v, didx_v, s0_v, d0_v, pbuf_v, outv_v,
                  ss0, sd0, ss1, sd1):
    nchunk = E // C
    bnk = nchunk // NW
    rem = nchunk - bnk * NW
    maxnk = bnk + (1 if rem else 0)
    wid = lax.axis_index("c") * 16 + lax.axis_index("s")
    nk = jnp.where(wid < rem, bnk + 1, bnk)
    start_chunk = wid * bnk + jnp.minimum(wid, rem)
    ebase = start_chunk * C

    pltpu.sync_copy(src_hbm.at[pl.ds(ebase, maxnk * C)], sidx_v)
    pltpu.sync_copy(dst_hbm.at[pl.ds(ebase, maxnk * C)], didx_v)

    def start(c):
        par = c % 2

        @pl.when(par == 0)
        def _():
            pltpu.async_copy(z_hbm.at[sidx_v.at[pl.ds(c * C, C)]],
                             s0_v.at[pl.ds(0, C)], ss0)
            pltpu.async_copy(z_hbm.at[didx_v.at[pl.ds(c * C, C)]],
                             d0_v.at[pl.ds(0, C)], sd0)

        @pl.when(par == 1)
        def _():
            pltpu.async_copy(z_hbm.at[sidx_v.at[pl.ds(c * C, C)]],
                             s0_v.at[pl.ds(C, C)], ss1)
            pltpu.async_copy(z_hbm.at[didx_v.at[pl.ds(c * C, C)]],
                             d0_v.at[pl.ds(C, C)], sd1)

    def wait(c):
        par = c % 2

        @pl.when(par == 0)
        def _():
            pltpu.make_async_copy(z_hbm.at[sidx_v.at[pl.ds(c * C, C)]],
                                  s0_v.at[pl.ds(0, C)], ss0).wait()
            pltpu.make_async_copy(z_hbm.at[didx_v.at[pl.ds(c * C, C)]],
                                  d0_v.at[pl.ds(0, C)], sd0).wait()

        @pl.when(par == 1)
        def _():
            pltpu.make_async_copy(z_hbm.at[sidx_v.at[pl.ds(c * C, C)]],
                                  s0_v.at[pl.ds(C, C)], ss1).wait()
            pltpu.make_async_copy(z_hbm.at[didx_v.at[pl.ds(c * C, C)]],
                                  d0_v.at[pl.ds(C, C)], sd1).wait()

    lanes = lax.broadcasted_iota(jnp.int32, (L,), 0)
    maskf = [jnp.where(lanes == e, 1.0, 0.0).astype(jnp.float32)
             for e in range(L)]

    def compute(c):
        rowb = (c % 2) * C

        def group_body(g, _):
            def edge_body(e, res):
                row = rowb + g * L + e
                acc = (s0_v[row, pl.ds(0, L)] * d0_v[row, pl.ds(0, L)])
                for i in range(1, DV):
                    acc = acc + (s0_v[row, pl.ds(i * L, L)]
                                 * d0_v[row, pl.ds(i * L, L)])
                pbuf_v[pl.ds(e * PB, L)] = acc
                for h in (8, 4, 2):
                    acc = acc + pbuf_v[pl.ds(e * PB + h, L)]
                    pbuf_v[pl.ds(e * PB, L)] = acc
                mask = jnp.where(lanes == e, 1.0, 0.0).astype(jnp.float32)
                return res + (pbuf_v[pl.ds((PB - 1) * e, L)]
                              + pbuf_v[pl.ds((PB - 1) * e + 1, L)]) * mask

            res = lax.fori_loop(0, L, edge_body,
                                jnp.zeros((L,), jnp.float32))
            outv_v[pl.ds(g * L, L)] = 1.0 / (1.0 + jnp.exp(-res))
            return 0

        lax.fori_loop(0, C // L, group_body, 0)
        pltpu.sync_copy(outv_v, out_hbm.at[pl.ds(ebase + c * C, C)])

    start(0)

    def pipe_body(c, _):
        @pl.when(c + 1 < nk)
        def _():
            start(c + 1)

        @pl.when(c < nk)
        def _():
            wait(c)
            compute(c)

        return 0

    lax.fori_loop(0, maxnk, pipe_body, 0)


def kernel(z, edge_index):
    E = edge_index.shape[1]
    nchunk = E // C
    bnk = nchunk // NW
    maxnk = bnk + (1 if nchunk % NW else 0)
    pad = maxnk * C * NW - E + C
    src = jnp.pad(edge_index[0], (0, pad))
    dst = jnp.pad(edge_index[1], (0, pad))

    mesh = plsc.VectorSubcoreMesh(core_axis_name="c", subcore_axis_name="s")
    body = functools.partial(_decoder_body, E)
    f = pl.kernel(
        body,
        out_type=jax.ShapeDtypeStruct((E,), jnp.float32),
        mesh=mesh,
        scratch_types=[
            pltpu.VMEM((maxnk * C,), jnp.int32),
            pltpu.VMEM((maxnk * C,), jnp.int32),
            pltpu.VMEM((2 * C, D), jnp.float32),
            pltpu.VMEM((2 * C, D), jnp.float32),
            pltpu.VMEM((L * PB + L,), jnp.float32),
            pltpu.VMEM((C,), jnp.float32),
            pltpu.SemaphoreType.DMA,
            pltpu.SemaphoreType.DMA,
            pltpu.SemaphoreType.DMA,
            pltpu.SemaphoreType.DMA,
        ],
    )
    return f(z, src, dst)


# speedup vs baseline: 2.0890x; 1.0017x over previous
import functools

import jax
import jax.numpy as jnp
from jax import lax
from jax.experimental import pallas as pl
from jax.experimental.pallas import tpu as pltpu
from jax.experimental.pallas import tpu_sc as plsc

L = 16
NW = 32
D = 256
DV = D // L
C = 64
PB = 32


def _decoder_body(E, z_hbm, src_hbm, dst_hbm, out_hbm,
                  sidx_v, didx_v, s0_v, d0_v, pbuf_v, outv_v,
                  ss0, sd0, ss1, sd1):
    nchunk = E // C
    bnk = nchunk // NW
    rem = nchunk - bnk * NW
    maxnk = bnk + (1 if rem else 0)
    wid = lax.axis_index("c") * 16 + lax.axis_index("s")
    nk = jnp.where(wid < rem, bnk + 1, bnk)
    start_chunk = wid * bnk + jnp.minimum(wid, rem)
    ebase = start_chunk * C

    pltpu.sync_copy(src_hbm.at[pl.ds(ebase, maxnk * C)], sidx_v)
    pltpu.sync_copy(dst_hbm.at[pl.ds(ebase, maxnk * C)], didx_v)

    def start(c):
        par = c % 2

        @pl.when(par == 0)
        def _():
            pltpu.async_copy(z_hbm.at[sidx_v.at[pl.ds(c * C, C)]],
                             s0_v.at[pl.ds(0, C)], ss0)
            pltpu.async_copy(z_hbm.at[didx_v.at[pl.ds(c * C, C)]],
                             d0_v.at[pl.ds(0, C)], sd0)

        @pl.when(par == 1)
        def _():
            pltpu.async_copy(z_hbm.at[sidx_v.at[pl.ds(c * C, C)]],
                             s0_v.at[pl.ds(C, C)], ss1)
            pltpu.async_copy(z_hbm.at[didx_v.at[pl.ds(c * C, C)]],
                             d0_v.at[pl.ds(C, C)], sd1)

    def wait(c):
        par = c % 2

        @pl.when(par == 0)
        def _():
            pltpu.make_async_copy(z_hbm.at[sidx_v.at[pl.ds(c * C, C)]],
                                  s0_v.at[pl.ds(0, C)], ss0).wait()
            pltpu.make_async_copy(z_hbm.at[didx_v.at[pl.ds(c * C, C)]],
                                  d0_v.at[pl.ds(0, C)], sd0).wait()

        @pl.when(par == 1)
        def _():
            pltpu.make_async_copy(z_hbm.at[sidx_v.at[pl.ds(c * C, C)]],
                                  s0_v.at[pl.ds(C, C)], ss1).wait()
            pltpu.make_async_copy(z_hbm.at[didx_v.at[pl.ds(c * C, C)]],
                                  d0_v.at[pl.ds(C, C)], sd1).wait()

    lanes = lax.broadcasted_iota(jnp.int32, (L,), 0)
    maskf = [jnp.where(lanes == e, 1.0, 0.0).astype(jnp.float32)
             for e in range(L)]

    def compute(c):
        rowb = (c % 2) * C

        def group_body(g, _):
            def edge_body(e, res):
                row = rowb + g * L + e
                acc = (s0_v[row, pl.ds(0, L)] * d0_v[row, pl.ds(0, L)])
                for i in range(1, DV):
                    acc = acc + (s0_v[row, pl.ds(i * L, L)]
                                 * d0_v[row, pl.ds(i * L, L)])
                pbuf_v[pl.ds(e * PB, L)] = acc
                for h in (8, 4, 2):
                    acc = acc + pbuf_v[pl.ds(e * PB + h, L)]
                    pbuf_v[pl.ds(e * PB, L)] = acc
                return jnp.where(lanes == e,
                                 pbuf_v[pl.ds((PB - 1) * e, L)]
                                 + pbuf_v[pl.ds((PB - 1) * e + 1, L)], res)

            res = lax.fori_loop(0, L, edge_body,
                                jnp.zeros((L,), jnp.float32))
            outv_v[pl.ds(g * L, L)] = 1.0 / (1.0 + jnp.exp(-res))
            return 0

        lax.fori_loop(0, C // L, group_body, 0)
        pltpu.sync_copy(outv_v, out_hbm.at[pl.ds(ebase + c * C, C)])

    start(0)

    def pipe_body(c, _):
        @pl.when(c + 1 < nk)
        def _():
            start(c + 1)

        @pl.when(c < nk)
        def _():
            wait(c)
            compute(c)

        return 0

    lax.fori_loop(0, maxnk, pipe_body, 0)


def kernel(z, edge_index):
    E = edge_index.shape[1]
    nchunk = E // C
    bnk = nchunk // NW
    maxnk = bnk + (1 if nchunk % NW else 0)
    pad = maxnk * C * NW - E + C
    src = jnp.pad(edge_index[0], (0, pad))
    dst = jnp.pad(edge_index[1], (0, pad))

    mesh = plsc.VectorSubcoreMesh(core_axis_name="c", subcore_axis_name="s")
    body = functools.partial(_decoder_body, E)
    f = pl.kernel(
        body,
        out_type=jax.ShapeDtypeStruct((E,), jnp.float32),
        mesh=mesh,
        scratch_types=[
            pltpu.VMEM((maxnk * C,), jnp.int32),
            pltpu.VMEM((maxnk * C,), jnp.int32),
            pltpu.VMEM((2 * C, D), jnp.float32),
            pltpu.VMEM((2 * C, D), jnp.float32),
            pltpu.VMEM((L * PB + L,), jnp.float32),
            pltpu.VMEM((C,), jnp.float32),
            pltpu.SemaphoreType.DMA,
            pltpu.SemaphoreType.DMA,
            pltpu.SemaphoreType.DMA,
            pltpu.SemaphoreType.DMA,
        ],
    )
    return f(z, src, dst)
